# baseline (speedup 1.0000x reference)
import functools

import jax
import jax.numpy as jnp
from jax import lax
from jax.experimental import pallas as pl
from jax.experimental.pallas import tpu as pltpu
from jax.experimental.pallas import tpu_sc as plsc

N_NODES = 10000
D_FEAT = 128
E_EDGES = 320000

NC, NS = 2, 16
NW = NC * NS
N_PAD = 10240
ROWS_PT = N_PAD // NS
BB = 128
E_PT = E_EDGES // NW
E_PAD_PT = 10240
NBLK_PT = E_PAD_PT // BB
KSB = 20
NSB = NBLK_PT // KSB

_sc_mesh = plsc.VectorSubcoreMesh(core_axis_name="c", subcore_axis_name="s")


@functools.partial(
    pl.kernel,
    mesh=_sc_mesh,
    out_type=(
        jax.ShapeDtypeStruct((NC, N_PAD, D_FEAT), jnp.float32),
        jax.ShapeDtypeStruct((NC, N_PAD), jnp.float32),
    ),
    scratch_types=[
        pltpu.VMEM((KSB, BB), jnp.int32),
        pltpu.VMEM((KSB, BB), jnp.int32),
        pltpu.VMEM((KSB, BB), jnp.int32),
        pltpu.VMEM((KSB, BB), jnp.int32),
        pltpu.VMEM((BB, D_FEAT), jnp.float32),
        pltpu.VMEM((BB, D_FEAT), jnp.float32),
        pltpu.VMEM((BB,), jnp.float32),
        pltpu.VMEM_SHARED((N_PAD, D_FEAT), jnp.float32),
        pltpu.VMEM_SHARED((N_PAD,), jnp.float32),
        pltpu.SemaphoreType.DMA,
        pltpu.SemaphoreType.DMA,
        pltpu.SemaphoreType.DMA,
    ],
)
def _scatter_sum_sc(emb_hbm, src_hbm, dst_hbm, zrow_hbm, zdeg_hbm,
                    sums_out, deg_out,
                    srcA_v, srcB_v, dstA_v, dstB_v, rows0_v, rows1_v, ones_v,
                    acc_sh, deg_sh, sem0, sem1, semi):
    c = lax.axis_index("c")
    s = lax.axis_index("s")
    wid = s * NC + c
    bufs = ((rows0_v, sem0), (rows1_v, sem1))
    sbufs = ((srcA_v, dstA_v), (srcB_v, dstB_v))

    for i in range(BB // 16):
        ones_v[pl.ds(i * 16, 16)] = jnp.ones((16,), jnp.float32)

    pltpu.sync_copy(zrow_hbm, acc_sh.at[pl.ds(s * ROWS_PT, ROWS_PT)])
    pltpu.sync_copy(zdeg_hbm, deg_sh.at[pl.ds(s * ROWS_PT, ROWS_PT)])

    pltpu.sync_copy(src_hbm.at[wid, 0], srcA_v)
    pltpu.sync_copy(dst_hbm.at[wid, 0], dstA_v)

    plsc.subcore_barrier()

    def _gather(idx, rows_v, sem):
        pltpu.async_copy(emb_hbm.at[idx], rows_v, sem)

    def _consume(src_v, dst_v, j, rows_v, sem):
        pltpu.make_async_copy(emb_hbm.at[src_v.at[j]], rows_v, sem).wait()
        pltpu.sync_copy(rows_v, acc_sh.at[dst_v.at[j]], add=True)
        pltpu.sync_copy(ones_v, deg_sh.at[dst_v.at[j]], add=True)

    for q in range(NSB):
        src_v, dst_v = sbufs[q % 2]
        src_n, dst_n = sbufs[(q + 1) % 2]
        if q + 1 < NSB:
            pltpu.async_copy(src_hbm.at[wid, q + 1], src_n, semi)
            pltpu.async_copy(dst_hbm.at[wid, q + 1], dst_n, semi)

        for b, (rows_v, sem) in enumerate(bufs):
            _gather(src_v.at[b], rows_v, sem)

        def pair(g, carry):
            for b, (rows_v, sem) in enumerate(bufs):
                j = 2 * g + b
                _consume(src_v, dst_v, j, rows_v, sem)
                _gather(src_v.at[j + 2], rows_v, sem)
            return carry

        lax.fori_loop(0, (KSB - 2) // 2, pair, 0)

        for b, (rows_v, sem) in enumerate(bufs):
            _consume(src_v, dst_v, KSB - 2 + b, rows_v, sem)

        if q + 1 < NSB:
            pltpu.make_async_copy(src_hbm.at[wid, q + 1], src_n, semi).wait()
            pltpu.make_async_copy(dst_hbm.at[wid, q + 1], dst_n, semi).wait()

    plsc.subcore_barrier()

    pltpu.sync_copy(acc_sh.at[pl.ds(s * ROWS_PT, ROWS_PT)],
                    sums_out.at[c, pl.ds(s * ROWS_PT, ROWS_PT)])
    pltpu.sync_copy(deg_sh.at[pl.ds(s * ROWS_PT, ROWS_PT)],
                    deg_out.at[c, pl.ds(s * ROWS_PT, ROWS_PT)])


def _combine_body(sums_ref, deg_ref, out_ref):
    t = sums_ref[0] + sums_ref[1]
    d = deg_ref[0] + deg_ref[1]
    dcol = d[:, None]
    out_ref[...] = jnp.where(dcol > 0, t / jnp.maximum(dcol, 1.0),
                             jnp.zeros_like(t))


_ROWS_BLK = 1024
_combine = pl.pallas_call(
    _combine_body,
    grid=(N_PAD // _ROWS_BLK,),
    in_specs=[
        pl.BlockSpec((NC, _ROWS_BLK, D_FEAT), lambda i: (0, i, 0)),
        pl.BlockSpec((NC, _ROWS_BLK), lambda i: (0, i)),
    ],
    out_specs=pl.BlockSpec((_ROWS_BLK, D_FEAT), lambda i: (i, 0)),
    out_shape=jax.ShapeDtypeStruct((N_NODES, D_FEAT), jnp.float32),
)


def kernel(embeddings, edge_index):
    src = edge_index[0].astype(jnp.int32).reshape(NW, E_PT)
    dst = edge_index[1].astype(jnp.int32).reshape(NW, E_PT)
    pad_e = E_PAD_PT - E_PT
    src = jnp.pad(src, ((0, 0), (0, pad_e)))
    dst = jnp.pad(dst, ((0, 0), (0, pad_e)),
                  constant_values=N_NODES)
    src = src.reshape(NW, NSB, KSB, BB)
    dst = dst.reshape(NW, NSB, KSB, BB)
    zrow = jnp.zeros((ROWS_PT, D_FEAT), jnp.float32)
    zdeg = jnp.zeros((ROWS_PT,), jnp.float32)
    sums, deg = _scatter_sum_sc(embeddings, src, dst, zrow, zdeg)
    return _combine(sums, deg)
